# baseline (speedup 1.0000x reference)
import jax
import jax.numpy as jnp
from jax import lax
from jax.experimental import pallas as pl
from jax.experimental.pallas import tpu as pltpu
from jax.experimental.pallas import tpu_sc as plsc

N_ATTENDED = 48
VALUE_BITS = 256
COUNT_BITS = 7
N_BITS_PER_NEURON = 14
TAIL_START = VALUE_BITS * COUNT_BITS
L = 16
NS = 16
NEUR_PER_SUB = VALUE_BITS // NS


def _addr_body(av_ref, conn_ref, idx_ref):
    counts = jnp.minimum(
        jnp.sum(av_ref[...], axis=0, dtype=jnp.int32), 64)
    conn = conn_ref[...]
    c = jnp.minimum(conn // COUNT_BITS, VALUE_BITS - 1)
    iota = lax.broadcasted_iota(jnp.int32, (VALUE_BITS, N_BITS_PER_NEURON,
                                            VALUE_BITS), 2)
    onehot = (iota == c[:, :, None]).astype(jnp.float32)
    cnt = jnp.sum(onehot * counts.astype(jnp.float32)[None, None, :],
                  axis=2).astype(jnp.int32)
    k = conn % COUNT_BITS
    bit_main = (cnt >> (6 - k)) & 1
    q = jnp.clip(conn - TAIL_START, 0, COUNT_BITS - 1)
    bit_tail = (N_ATTENDED >> (6 - q)) & 1
    bit = jnp.where(conn < TAIL_START, bit_main, bit_tail)
    powers = (1 << jnp.arange(N_BITS_PER_NEURON - 1, -1, -1,
                              dtype=jnp.int32))
    addr = jnp.sum(bit * powers[None, :], axis=1)
    n = lax.broadcasted_iota(jnp.int32, (VALUE_BITS,), 0)
    idx_ref[...] = (((n >> 3) * 128 + (addr >> 7)) << 10) \
        | ((n & 7) << 7) | (addr & 127)


def _gather_body(idx_hbm, ram_hbm, out_hbm, idx_v, vals_v, sem):
    s = lax.axis_index("s")
    n0 = pl.multiple_of(s * NEUR_PER_SUB, NEUR_PER_SUB)
    pltpu.sync_copy(idx_hbm.at[pl.ds(n0, NEUR_PER_SUB)], idx_v)
    pltpu.async_copy(ram_hbm.at[idx_v], vals_v, sem).wait()
    pltpu.sync_copy(vals_v, out_hbm.at[pl.ds(n0, NEUR_PER_SUB)])


@jax.jit
def kernel(attended_values, connections, ram_table):
    ram_flat = jnp.transpose(
        ram_table.reshape(32, 8, 128, 128), (0, 2, 1, 3)).reshape(-1)

    idx = pl.pallas_call(
        _addr_body,
        out_shape=jax.ShapeDtypeStruct((VALUE_BITS,), jnp.int32),
    )(attended_values, connections)

    mesh = plsc.VectorSubcoreMesh(
        core_axis_name="c", subcore_axis_name="s", num_cores=1)
    f = pl.kernel(
        _gather_body,
        out_type=jax.ShapeDtypeStruct((VALUE_BITS,), jnp.float32),
        mesh=mesh,
        scratch_types=[
            pltpu.VMEM((L,), jnp.int32),
            pltpu.VMEM((L,), jnp.float32),
            pltpu.SemaphoreType.DMA,
        ],
        compiler_params=pltpu.CompilerParams(
            use_tc_tiling_on_sc=False, needs_layout_passes=False),
    )
    return f(idx, ram_flat)


# speedup vs baseline: 1.0008x; 1.0008x over previous
import jax
import jax.numpy as jnp
from jax import lax
from jax.experimental import pallas as pl
from jax.experimental.pallas import tpu as pltpu
from jax.experimental.pallas import tpu_sc as plsc

N_ATTENDED = 48
VALUE_BITS = 256
COUNT_BITS = 7
N_BITS_PER_NEURON = 14
TAIL_START = VALUE_BITS * COUNT_BITS
L = 16
NS = 16
NEUR_PER_SUB = VALUE_BITS // NS


def _addr_body(av_ref, conn_ref, idx_ref):
    counts = jnp.minimum(
        jnp.sum(av_ref[...], axis=0, dtype=jnp.int32), 64)
    conn = conn_ref[...]
    c = jnp.minimum(conn // COUNT_BITS, VALUE_BITS - 1)
    iota = lax.broadcasted_iota(jnp.int32, (VALUE_BITS, N_BITS_PER_NEURON,
                                            VALUE_BITS), 2)
    onehot = jnp.where(iota == c[:, :, None],
                       counts.astype(jnp.float32)[None, None, :],
                       jnp.float32(0))
    cnt = jnp.sum(onehot, axis=2).astype(jnp.int32)
    k = conn % COUNT_BITS
    bit_main = (cnt >> (6 - k)) & 1
    q = jnp.clip(conn - TAIL_START, 0, COUNT_BITS - 1)
    bit_tail = (N_ATTENDED >> (6 - q)) & 1
    bit = jnp.where(conn < TAIL_START, bit_main, bit_tail)
    powers = (1 << jnp.arange(N_BITS_PER_NEURON - 1, -1, -1,
                              dtype=jnp.int32))
    addr = jnp.sum(bit * powers[None, :], axis=1)
    n = lax.broadcasted_iota(jnp.int32, (VALUE_BITS,), 0)
    idx_ref[...] = (((n >> 3) * 128 + (addr >> 7)) << 10) \
        | ((n & 7) << 7) | (addr & 127)


def _gather_body(idx_hbm, ram_hbm, out_hbm, idx_v, vals_v, sem):
    s = lax.axis_index("s")
    n0 = pl.multiple_of(s * NEUR_PER_SUB, NEUR_PER_SUB)
    pltpu.sync_copy(idx_hbm.at[pl.ds(n0, NEUR_PER_SUB)], idx_v)
    pltpu.async_copy(ram_hbm.at[idx_v], vals_v, sem).wait()
    pltpu.sync_copy(vals_v, out_hbm.at[pl.ds(n0, NEUR_PER_SUB)])


@jax.jit
def kernel(attended_values, connections, ram_table):
    ram_flat = jnp.transpose(
        ram_table.reshape(32, 8, 128, 128), (0, 2, 1, 3)).reshape(-1)

    idx = pl.pallas_call(
        _addr_body,
        out_shape=jax.ShapeDtypeStruct((VALUE_BITS,), jnp.int32),
    )(attended_values, connections)

    mesh = plsc.VectorSubcoreMesh(
        core_axis_name="c", subcore_axis_name="s", num_cores=1)
    f = pl.kernel(
        _gather_body,
        out_type=jax.ShapeDtypeStruct((VALUE_BITS,), jnp.float32),
        mesh=mesh,
        scratch_types=[
            pltpu.VMEM((L,), jnp.int32),
            pltpu.VMEM((L,), jnp.float32),
            pltpu.SemaphoreType.DMA,
        ],
        compiler_params=pltpu.CompilerParams(
            use_tc_tiling_on_sc=False, needs_layout_passes=False),
    )
    return f(idx, ram_flat)


# speedup vs baseline: 1.0472x; 1.0464x over previous
import jax
import jax.numpy as jnp
from jax import lax
from jax.experimental import pallas as pl
from jax.experimental.pallas import tpu as pltpu
from jax.experimental.pallas import tpu_sc as plsc

N_ATTENDED = 48
VALUE_BITS = 256
COUNT_BITS = 7
N_BITS_PER_NEURON = 14
RAM_SIZE = 16384
TAIL_START = VALUE_BITS * COUNT_BITS
L = 16
NS = 16
ROWS_PER_SUM = 8
NSUM = N_ATTENDED // ROWS_PER_SUM
NEUR_PER_SUB = VALUE_BITS // NS


def _body(av_hbm, conn_hbm, ram_hbm, out_hbm,
          av_v, conn_v, part_v, csum_v, counts_v, idx_v, vals_v,
          sh_counts, sem, conn_sem):
    s = lax.axis_index("s")

    c0 = pl.multiple_of(s * (NEUR_PER_SUB * N_BITS_PER_NEURON),
                        NEUR_PER_SUB * N_BITS_PER_NEURON)
    conn_dma = pltpu.async_copy(
        conn_hbm.at[pl.ds(c0, NEUR_PER_SUB * N_BITS_PER_NEURON)],
        conn_v, conn_sem)

    @pl.when(s < 2 * NSUM)
    def _():
        pltpu.sync_copy(av_hbm.at[s >> 1, s & 1], av_v)

        def p1(k, _):
            sl = pl.ds(k * L, L)
            acc = av_v[0, sl]
            for r in range(1, ROWS_PER_SUM):
                acc = acc + av_v[r, sl]
            part_v[sl] = acc
            return 0

        lax.fori_loop(0, 128 // L, p1, 0)
        pltpu.sync_copy(part_v, sh_counts.at[s])

    plsc.subcore_barrier()
    pltpu.sync_copy(sh_counts, csum_v)

    def merge(t, _):
        ch = t >> 3
        sl = pl.ds((t & 7) * L, L)
        tot = csum_v[ch, sl]
        for rb in range(1, NSUM):
            tot = tot + csum_v[2 * rb + ch, sl]
        counts_v[pl.ds(t * L, L)] = jnp.minimum(tot, 64)
        return 0

    lax.fori_loop(0, VALUE_BITS // L, merge, 0)

    n0 = pl.multiple_of(s * NEUR_PER_SUB, NEUR_PER_SUB)
    conn_dma.wait()
    lanes = lax.broadcasted_iota(jnp.int32, (L,), 0)
    conn_base = lanes * N_BITS_PER_NEURON

    def p2(j, addr):
        p = plsc.load_gather(conn_v, [conn_base + j])
        c = jnp.minimum(p // COUNT_BITS, VALUE_BITS - 1)
        k = p % COUNT_BITS
        cnt = plsc.load_gather(counts_v, [c])
        bit_main = (cnt >> (6 - k)) & 1
        q = jnp.clip(p - TAIL_START, 0, COUNT_BITS - 1)
        bit_tail = (jnp.full((L,), N_ATTENDED, jnp.int32) >> (6 - q)) & 1
        bit = jnp.where(p < TAIL_START, bit_main, bit_tail)
        return (addr << 1) | bit

    addr = lax.fori_loop(0, N_BITS_PER_NEURON, p2, jnp.zeros((L,), jnp.int32))
    n = n0 + lanes
    idx_v[...] = (((n >> 3) * 128 + (addr >> 7)) << 10) \
        | ((n & 7) << 7) | (addr & 127)
    pltpu.async_copy(ram_hbm.at[idx_v], vals_v, sem).wait()
    pltpu.sync_copy(vals_v, out_hbm.at[pl.ds(n0, NEUR_PER_SUB)])


@jax.jit
def kernel(attended_values, connections, ram_table):
    ram_flat = jnp.transpose(
        ram_table.reshape(32, 8, 128, 128), (0, 2, 1, 3)).reshape(-1)
    av4 = jnp.transpose(
        attended_values.reshape(6, 8, 2, 128), (0, 2, 1, 3))
    conn_flat = connections.reshape(-1)
    mesh = plsc.VectorSubcoreMesh(
        core_axis_name="c", subcore_axis_name="s", num_cores=1)
    f = pl.kernel(
        _body,
        out_type=jax.ShapeDtypeStruct((VALUE_BITS,), jnp.float32),
        mesh=mesh,
        scratch_types=[
            pltpu.VMEM((ROWS_PER_SUM, 128), jnp.int32),
            pltpu.VMEM((NEUR_PER_SUB * N_BITS_PER_NEURON,), jnp.int32),
            pltpu.VMEM((128,), jnp.int32),
            pltpu.VMEM((2 * NSUM, 128), jnp.int32),
            pltpu.VMEM((VALUE_BITS,), jnp.int32),
            pltpu.VMEM((L,), jnp.int32),
            pltpu.VMEM((L,), jnp.float32),
            pltpu.VMEM_SHARED((2 * NSUM, 128), jnp.int32),
            pltpu.SemaphoreType.DMA,
            pltpu.SemaphoreType.DMA,
        ],
        compiler_params=pltpu.CompilerParams(
            use_tc_tiling_on_sc=False, needs_layout_passes=False),
    )
    return f(av4, conn_flat, ram_flat)
